# baseline (speedup 1.0000x reference)
import functools

import jax
import jax.numpy as jnp
from jax import lax
from jax.experimental import pallas as pl
from jax.experimental.pallas import tpu as pltpu
from jax.experimental.pallas import tpu_sc as plsc

_B, _C, _H, _W = 8, 19, 512, 512
_HW = _H * _W
_NW = 32
_L = 16
_TS = 8192
_VECS = _TS // _L

_KSC = 3
_B0 = _B - _KSC
_PER_W = _KSC * _HW // _NW
_NT = _PER_W // _TS

_BH = 128


def _sc_body(z_hbm, l_hbm, lt_hbm, cm_hbm, out_hbm,
             l_v0, l_v1, lt_v0, lt_v1, cm_v0, cm_v1,
             ig_v0, ig_v1, ib_v0, ib_v1, vg_v0, vg_v1, vb_v0, vb_v1,
             acc_v,
             sem_i0, sem_i1, sem_g0, sem_g1, sem_b0, sem_b1):
  lv = (l_v0, l_v1)
  ltv = (lt_v0, lt_v1)
  cmv = (cm_v0, cm_v1)
  igv = (ig_v0, ig_v1)
  ibv = (ib_v0, ib_v1)
  vgv = (vg_v0, vg_v1)
  vbv = (vb_v0, vb_v1)
  sin = (sem_i0, sem_i1)
  sg = (sem_g0, sem_g1)
  sb = (sem_b0, sem_b1)

  cid = lax.axis_index("c")
  sid = lax.axis_index("s")
  wid = sid * 2 + cid
  base = _B0 * _HW + wid * _PER_W

  def in_copies(t, s):
    st = base + t * _TS
    return (
        pltpu.make_async_copy(l_hbm.at[pl.ds(st, _TS)], lv[s], sin[s]),
        pltpu.make_async_copy(lt_hbm.at[pl.ds(st, _TS)], ltv[s], sin[s]),
        pltpu.make_async_copy(cm_hbm.at[pl.ds(st, _TS)], cmv[s], sin[s]),
    )

  def gather_copies(s):
    return (
        pltpu.make_async_copy(z_hbm.at[igv[s]], vgv[s], sg[s]),
        pltpu.make_async_copy(z_hbm.at[ibv[s]], vbv[s], sb[s]),
    )

  def build(t, s):
    g0 = base + t * _TS
    b = g0 // _HW
    tilebase = g0 + b * (_C - 1) * _HW
    l_r, lt_r, cm_r, ig_r, ib_r = lv[s], ltv[s], cmv[s], igv[s], ibv[s]

    def vec(j, c):
      off = j * _L
      qv = lax.iota(jnp.int32, _L) + (tilebase + off)
      ib = lt_r[pl.ds(off, _L)] * _HW + qv
      ig = l_r[pl.ds(off, _L)] * _HW + qv
      m = cm_r[pl.ds(off, _L)] != 0
      ig_r[pl.ds(off, _L)] = jnp.where(m, ig, ib)
      ib_r[pl.ds(off, _L)] = ib
      return c
    lax.fori_loop(0, _VECS, vec, 0, unroll=8)

  def accum(s, acc):
    vg_r, vb_r = vgv[s], vbv[s]

    def vec(j, a):
      off = j * _L
      return a + (vg_r[pl.ds(off, _L)] - vb_r[pl.ds(off, _L)])
    return lax.fori_loop(0, _VECS, vec, acc, unroll=8)

  for c in in_copies(0, 0):
    c.start()
  if _NT > 1:
    for c in in_copies(1, 1):
      c.start()

  acc = jnp.zeros((_L,), jnp.float32)
  for t in range(_NT):
    s = t % 2
    for c in in_copies(t, s):
      c.wait()
    build(t, s)
    for c in gather_copies(s):
      c.start()
    if t + 2 < _NT:
      for c in in_copies(t + 2, s):
        c.start()
    if t >= 1:
      for c in gather_copies(1 - s):
        c.wait()
      acc = accum(1 - s, acc)
  s_last = (_NT - 1) % 2
  for c in gather_copies(s_last):
    c.wait()
  acc = accum(s_last, acc)

  acc_v[...] = acc
  pltpu.sync_copy(acc_v, out_hbm.at[wid])


def _make_sc_kernel():
  mesh = plsc.VectorSubcoreMesh(core_axis_name="c", subcore_axis_name="s")
  buf_i32 = pltpu.VMEM((_TS,), jnp.int32)
  buf_f32 = pltpu.VMEM((_TS,), jnp.float32)
  return pl.kernel(
      _sc_body,
      out_type=jax.ShapeDtypeStruct((_NW, _L), jnp.float32),
      mesh=mesh,
      scratch_types=[
          buf_i32, buf_i32,
          buf_i32, buf_i32,
          buf_i32, buf_i32,
          buf_i32, buf_i32,
          buf_i32, buf_i32,
          buf_f32, buf_f32,
          buf_f32, buf_f32,
          pltpu.VMEM((_L,), jnp.float32),
          pltpu.SemaphoreType.DMA,
          pltpu.SemaphoreType.DMA,
          pltpu.SemaphoreType.DMA,
          pltpu.SemaphoreType.DMA,
          pltpu.SemaphoreType.DMA,
          pltpu.SemaphoreType.DMA,
      ],
  )


def _tc_body(z_ref, l_ref, lt_ref, cm_ref, out_ref):
  @pl.when((pl.program_id(0) == 0) & (pl.program_id(1) == 0))
  def _init():
    out_ref[...] = jnp.zeros_like(out_ref)

  m = cm_ref[0] != 0
  ln = jnp.where(m, l_ref[0], -1)
  ltn = jnp.where(m, lt_ref[0], -1)
  acc = jnp.zeros((_BH, _W), jnp.float32)
  for c in range(_C):
    zc = z_ref[0, c]
    acc = acc + jnp.where(ln == c, zc, 0.0) - jnp.where(ltn == c, zc, 0.0)
  part = jnp.sum(acc.reshape(_BH // 8, 8, _W), axis=0)
  part = jnp.sum(part.reshape(8, _W // 128, 128), axis=1)
  out_ref[...] += part


def _make_tc_kernel():
  nbh = _H // _BH
  return pl.pallas_call(
      _tc_body,
      grid=(_B0, nbh),
      in_specs=[
          pl.BlockSpec((1, _C, _BH, _W), lambda i, j: (i, 0, j, 0)),
          pl.BlockSpec((1, _BH, _W), lambda i, j: (i, j, 0)),
          pl.BlockSpec((1, _BH, _W), lambda i, j: (i, j, 0)),
          pl.BlockSpec((1, _BH, _W), lambda i, j: (i, j, 0)),
      ],
      out_specs=pl.BlockSpec((8, 128), lambda i, j: (0, 0)),
      out_shape=jax.ShapeDtypeStruct((8, 128), jnp.float32),
  )


def _flat_tiled_plane(x):
  return x.reshape(8, 64, 8, 4, 128).transpose(0, 1, 3, 2, 4).reshape(-1)


def _flat_tiled_z(z):
  return z.reshape(8, 19, 64, 8, 4, 128).transpose(0, 1, 2, 4, 3, 5).reshape(-1)


def kernel(z, condition, l, l_target):
  li = l.astype(jnp.int32)
  lti = l_target.astype(jnp.int32)
  cmi = condition.astype(jnp.int32)
  zf = _flat_tiled_z(z)
  lf = _flat_tiled_plane(li)
  ltf = _flat_tiled_plane(lti)
  cmf = _flat_tiled_plane(cmi)
  sc_partials = _make_sc_kernel()(zf, lf, ltf, cmf)
  tc_partials = _make_tc_kernel()(z, li, lti, cmi)
  return jnp.sum(sc_partials) + jnp.sum(tc_partials)


# speedup vs baseline: 1.1182x; 1.1182x over previous
import functools

import jax
import jax.numpy as jnp
from jax import lax
from jax.experimental import pallas as pl
from jax.experimental.pallas import tpu as pltpu
from jax.experimental.pallas import tpu_sc as plsc

_B, _C, _H, _W = 8, 19, 512, 512
_HW = _H * _W
_NW = 32
_L = 16
_TS = 8192
_VECS = _TS // _L

_KSC = 2
_B0 = _B - _KSC
_PER_W = _KSC * _HW // _NW
_NT = _PER_W // _TS

_BH = 128


def _sc_body(z_hbm, l_hbm, lt_hbm, cm_hbm, out_hbm,
             l_v0, l_v1, lt_v0, lt_v1, cm_v0, cm_v1,
             ig_v0, ig_v1, ib_v0, ib_v1, vg_v0, vg_v1, vb_v0, vb_v1,
             acc_v,
             sem_i0, sem_i1, sem_g0, sem_g1, sem_b0, sem_b1):
  lv = (l_v0, l_v1)
  ltv = (lt_v0, lt_v1)
  cmv = (cm_v0, cm_v1)
  igv = (ig_v0, ig_v1)
  ibv = (ib_v0, ib_v1)
  vgv = (vg_v0, vg_v1)
  vbv = (vb_v0, vb_v1)
  sin = (sem_i0, sem_i1)
  sg = (sem_g0, sem_g1)
  sb = (sem_b0, sem_b1)

  cid = lax.axis_index("c")
  sid = lax.axis_index("s")
  wid = sid * 2 + cid
  base = _B0 * _HW + wid * _PER_W

  def in_copies(t, s):
    st = base + t * _TS
    stl = wid * _PER_W + t * _TS
    return (
        pltpu.make_async_copy(l_hbm.at[pl.ds(st, _TS)], lv[s], sin[s]),
        pltpu.make_async_copy(lt_hbm.at[pl.ds(st, _TS)], ltv[s], sin[s]),
        pltpu.make_async_copy(cm_hbm.at[pl.ds(stl, _TS)], cmv[s], sin[s]),
    )

  def gather_copies(s):
    return (
        pltpu.make_async_copy(z_hbm.at[igv[s]], vgv[s], sg[s]),
        pltpu.make_async_copy(z_hbm.at[ibv[s]], vbv[s], sb[s]),
    )

  def build(t, s):
    g0 = base + t * _TS
    b = g0 // _HW
    tilebase = g0 + b * (_C - 1) * _HW
    l_r, lt_r, cm_r, ig_r, ib_r = lv[s], ltv[s], cmv[s], igv[s], ibv[s]

    def vec(j, c):
      off = j * _L
      qv = lax.iota(jnp.int32, _L) + (tilebase + off)
      ib = lt_r[pl.ds(off, _L)] * _HW + qv
      ig = l_r[pl.ds(off, _L)] * _HW + qv
      m = cm_r[pl.ds(off, _L)] != 0
      ig_r[pl.ds(off, _L)] = jnp.where(m, ig, ib)
      ib_r[pl.ds(off, _L)] = ib
      return c
    lax.fori_loop(0, _VECS, vec, 0, unroll=8)

  def accum(s, acc):
    vg_r, vb_r = vgv[s], vbv[s]

    def vec(j, a):
      off = j * _L
      return a + (vg_r[pl.ds(off, _L)] - vb_r[pl.ds(off, _L)])
    return lax.fori_loop(0, _VECS, vec, acc, unroll=8)

  for c in in_copies(0, 0):
    c.start()
  if _NT > 1:
    for c in in_copies(1, 1):
      c.start()

  acc = jnp.zeros((_L,), jnp.float32)
  for t in range(_NT):
    s = t % 2
    for c in in_copies(t, s):
      c.wait()
    build(t, s)
    for c in gather_copies(s):
      c.start()
    if t + 2 < _NT:
      for c in in_copies(t + 2, s):
        c.start()
    if t >= 1:
      for c in gather_copies(1 - s):
        c.wait()
      acc = accum(1 - s, acc)
  s_last = (_NT - 1) % 2
  for c in gather_copies(s_last):
    c.wait()
  acc = accum(s_last, acc)

  acc_v[...] = acc
  pltpu.sync_copy(acc_v, out_hbm.at[wid])


def _make_sc_kernel():
  mesh = plsc.VectorSubcoreMesh(core_axis_name="c", subcore_axis_name="s")
  buf_i32 = pltpu.VMEM((_TS,), jnp.int32)
  buf_f32 = pltpu.VMEM((_TS,), jnp.float32)
  return pl.kernel(
      _sc_body,
      out_type=jax.ShapeDtypeStruct((_NW, _L), jnp.float32),
      mesh=mesh,
      scratch_types=[
          buf_i32, buf_i32,
          buf_i32, buf_i32,
          buf_i32, buf_i32,
          buf_i32, buf_i32,
          buf_i32, buf_i32,
          buf_f32, buf_f32,
          buf_f32, buf_f32,
          pltpu.VMEM((_L,), jnp.float32),
          pltpu.SemaphoreType.DMA,
          pltpu.SemaphoreType.DMA,
          pltpu.SemaphoreType.DMA,
          pltpu.SemaphoreType.DMA,
          pltpu.SemaphoreType.DMA,
          pltpu.SemaphoreType.DMA,
      ],
  )


def _tc_body(z_ref, l_ref, lt_ref, cm_ref, out_ref):
  @pl.when((pl.program_id(0) == 0) & (pl.program_id(1) == 0))
  def _init():
    out_ref[...] = jnp.zeros_like(out_ref)

  m = cm_ref[0]
  ln = jnp.where(m, l_ref[0], -1)
  ltn = jnp.where(m, lt_ref[0], -1)
  acc = jnp.zeros((_BH, _W), jnp.float32)
  for c in range(_C):
    zc = z_ref[0, c]
    acc = acc + jnp.where(ln == c, zc, 0.0) - jnp.where(ltn == c, zc, 0.0)
  part = jnp.sum(acc.reshape(_BH // 8, 8, _W), axis=0)
  part = jnp.sum(part.reshape(8, _W // 128, 128), axis=1)
  out_ref[...] += part


def _make_tc_kernel():
  nbh = _H // _BH
  return pl.pallas_call(
      _tc_body,
      grid=(_B0, nbh),
      in_specs=[
          pl.BlockSpec((1, _C, _BH, _W), lambda i, j: (i, 0, j, 0)),
          pl.BlockSpec((1, _BH, _W), lambda i, j: (i, j, 0)),
          pl.BlockSpec((1, _BH, _W), lambda i, j: (i, j, 0)),
          pl.BlockSpec((1, _BH, _W), lambda i, j: (i, j, 0)),
      ],
      out_specs=pl.BlockSpec((8, 128), lambda i, j: (0, 0)),
      out_shape=jax.ShapeDtypeStruct((8, 128), jnp.float32),
  )


def _flat_tiled_plane(x):
  return x.reshape(8, 64, 8, 4, 128).transpose(0, 1, 3, 2, 4).reshape(-1)


def _flat_tiled_z(z):
  return z.reshape(8, 19, 64, 8, 4, 128).transpose(0, 1, 2, 4, 3, 5).reshape(-1)


def _flat_tiled_sc_plane(x):
  return x.reshape(_KSC, 64, 8, 4, 128).transpose(0, 1, 3, 2, 4).reshape(-1)


def kernel(z, condition, l, l_target):
  li = l.astype(jnp.int32)
  lti = l_target.astype(jnp.int32)
  cmi_sc = condition[_B0:].astype(jnp.int32)
  zf = _flat_tiled_z(z)
  lf = _flat_tiled_plane(li)
  ltf = _flat_tiled_plane(lti)
  cmf = _flat_tiled_sc_plane(cmi_sc)
  sc_partials = _make_sc_kernel()(zf, lf, ltf, cmf)
  tc_partials = _make_tc_kernel()(z, li, lti, condition)
  return jnp.sum(sc_partials) + jnp.sum(tc_partials)
